# baseline (speedup 1.0000x reference)
import functools

import jax
import jax.numpy as jnp
from jax import lax
from jax.experimental import pallas as pl
from jax.experimental.pallas import tpu as pltpu
from jax.experimental.pallas import tpu_sc as plsc

_H, _W, _FDIM = 1024, 1024, 64
_HIDDEN, _OUT = 128, 64
_B, _N = 16, 65536
_M = _B * _N

_NC, _NS, _L = 2, 16, 16
_NW = _NC * _NS
_PW = _M // _NW
_BLK = 128
_NB = _PW // _BLK

_mesh = plsc.VectorSubcoreMesh(core_axis_name="c", subcore_axis_name="s")


def _make_sc_sample(m_pts):
  pw = m_pts // _NW
  nb = pw // _BLK

  @functools.partial(
    pl.kernel,
    mesh=_mesh,
    compiler_params=pltpu.CompilerParams(use_tc_tiling_on_sc=False),
    out_type=jax.ShapeDtypeStruct((m_pts // 2, 2 * _FDIM), jnp.float32),
    scratch_types=[
        pltpu.VMEM((2, _BLK), jnp.float32),
        pltpu.VMEM((2, _BLK), jnp.float32),
        pltpu.VMEM((2, 4, _BLK), jnp.int32),
        pltpu.VMEM((2, 4, _BLK), jnp.float32),
        pltpu.VMEM((2, 4 * _BLK, _FDIM), jnp.float32),
        pltpu.VMEM((2, _BLK, _FDIM), jnp.float32),
        pltpu.SemaphoreType.DMA,
        pltpu.SemaphoreType.DMA,
        pltpu.SemaphoreType.DMA,
        pltpu.SemaphoreType.DMA,
        pltpu.SemaphoreType.DMA,
        pltpu.SemaphoreType.DMA,
    ],
)
  def _sc_sample(xs_hbm, ys_hbm, table_hbm, out_hbm,
                 xs_v, ys_v, idx_v, w_v, rows_v, feats_v,
                 semc0, semc1, semg0, semg1, sems0, sems1):
      wid = lax.axis_index("s") * _NC + lax.axis_index("c")
      base = wid * pw
      semc = (semc0, semc1)
      semg = (semg0, semg1)
      sems = (sems0, sems1)

      def fire_coords(bi, par):
          off = base + bi * _BLK
          pltpu.async_copy(xs_hbm.at[pl.ds(off, _BLK)], xs_v.at[par], semc[par])
          pltpu.async_copy(ys_hbm.at[pl.ds(off, _BLK)], ys_v.at[par], semc[par])

      def wait_coords(par):
          pltpu.make_async_copy(
              xs_hbm.at[pl.ds(0, _BLK)], xs_v.at[par], semc[par]).wait()
          pltpu.make_async_copy(
              ys_hbm.at[pl.ds(0, _BLK)], ys_v.at[par], semc[par]).wait()

      def compute_idx(par):
          for j in range(_BLK // _L):
              sl = pl.ds(j * _L, _L)
              x = xs_v[par, sl] * jnp.float32(_W - 1)
              y = ys_v[par, sl] * jnp.float32(_H - 1)
              xi = jnp.minimum(x.astype(jnp.int32), _W - 2)
              yi = jnp.minimum(y.astype(jnp.int32), _H - 2)
              fx = x - xi.astype(jnp.float32)
              fy = y - yi.astype(jnp.float32)
              gx = 1.0 - fx
              gy = 1.0 - fy
              b00 = yi * _W + xi
              s0 = jnp.where(yi >= _H // 2, _H * _W - 1, 0)
              s1 = jnp.where(yi >= _H // 2 - 1, _H * _W - 1, 0)
              r00 = 2 * b00 - s0
              r10 = 2 * (b00 + _W) - s1
              idx_v[par, 0, sl] = r00
              idx_v[par, 1, sl] = r00 + 2
              idx_v[par, 2, sl] = r10
              idx_v[par, 3, sl] = r10 + 2
              w_v[par, 0, sl] = gx * gy
              w_v[par, 1, sl] = fx * gy
              w_v[par, 2, sl] = gx * fy
              w_v[par, 3, sl] = fx * fy

      def fire_gather(par):
          for c in range(4):
              pltpu.async_copy(table_hbm.at[idx_v.at[par, c]],
                               rows_v.at[par, pl.ds(c * _BLK, _BLK)], semg[par])

      def wait_gather(par):
          for c in range(4):
              pltpu.make_async_copy(
                  table_hbm.at[idx_v.at[par, c]],
                  rows_v.at[par, pl.ds(c * _BLK, _BLK)], semg[par]).wait()

      def blend(par):
          def group_body(g, c2):
              p0 = g * _L
              w00v = w_v[par, 0, pl.ds(p0, _L)]
              w01v = w_v[par, 1, pl.ds(p0, _L)]
              w10v = w_v[par, 2, pl.ds(p0, _L)]
              w11v = w_v[par, 3, pl.ds(p0, _L)]
              for i in range(_L):
                  p = p0 + i
                  w00 = w00v[i]
                  w01 = w01v[i]
                  w10 = w10v[i]
                  w11 = w11v[i]
                  for k in range(_FDIM // _L):
                      ksl = pl.ds(k * _L, _L)
                      acc = rows_v[par, p, ksl] * w00
                      acc = acc + rows_v[par, _BLK + p, ksl] * w01
                      acc = acc + rows_v[par, 2 * _BLK + p, ksl] * w10
                      acc = acc + rows_v[par, 3 * _BLK + p, ksl] * w11
                      feats_v[par, p, ksl] = acc
              return c2

          lax.fori_loop(0, _BLK // _L, group_body, 0)

      def fire_store(bi, par):
          off = base + bi * _BLK
          row_base = (off // 2048) * 1024 + lax.rem(off, 1024)
          half = lax.rem(off // 1024, 2)
          pltpu.async_copy(
              feats_v.at[par],
              out_hbm.at[pl.ds(row_base, _BLK), pl.ds(half * _FDIM, _FDIM)],
              sems[par])

      def wait_store(par):
          pltpu.make_async_copy(
              feats_v.at[par],
              out_hbm.at[pl.ds(0, _BLK), pl.ds(0, _FDIM)], sems[par]).wait()

      fire_coords(0, 0)
      fire_coords(1, 1)
      wait_coords(0)
      compute_idx(0)
      fire_gather(0)
      fire_coords(2, 0)
      wait_coords(1)
      compute_idx(1)
      fire_gather(1)
      fire_coords(3, 1)

      def outer(g, carry):
          for par in (0, 1):
              bi = 2 * g + par
              wait_gather(par)

              @pl.when(bi >= 2)
              def _():
                  wait_store(par)

              blend(par)
              fire_store(bi, par)

              @pl.when(bi + 2 < nb)
              def _():
                  wait_coords(par)
                  compute_idx(par)
                  fire_gather(par)

              @pl.when(bi + 4 < nb)
              def _():
                  fire_coords(bi + 4, par)

          return carry

      lax.fori_loop(0, nb // 2, outer, 0)
      wait_store(0)
      wait_store(1)

  return _sc_sample


_sc_sample_half = _make_sc_sample(_M // 2)


def _prep_body(ff_lo_ref, ff_hi_ref, out_ref):
    out_ref[0, :, pl.ds(0, _FDIM)] = ff_lo_ref[0].T
    out_ref[0, :, pl.ds(_FDIM, _FDIM)] = ff_hi_ref[0].T


def _prep(ff_t):
    return pl.pallas_call(
        _prep_body,
        grid=(_H // 2,),
        in_specs=[
            pl.BlockSpec((1, _FDIM, _W), lambda y: (y, 0, 0)),
            pl.BlockSpec((1, _FDIM, _W), lambda y: (y + _H // 2, 0, 0)),
        ],
        out_specs=pl.BlockSpec((1, _W, 2 * _FDIM), lambda y: (y, 0, 0)),
        out_shape=jax.ShapeDtypeStruct((_H // 2, _W, 2 * _FDIM), jnp.float32),
    )(ff_t, ff_t)


_BM = 2048


def _mlp_body(feats_ref, w1d_ref, b1d_ref, w2dt_ref, b2t_ref, out_ref):
    f2 = feats_ref[0].astype(jnp.bfloat16)
    h = jnp.dot(f2, w1d_ref[...],
                preferred_element_type=jnp.float32) + b1d_ref[...]
    h = jnp.maximum(h, 0.0).astype(jnp.bfloat16)
    o_t = lax.dot_general(w2dt_ref[...], h, (((1,), (1,)), ((), ())),
                          preferred_element_type=jnp.float32)
    b2t = b2t_ref[...]
    out_ref[0, :, pl.ds(0, _BM // 2)] = o_t[0:_OUT] + b2t
    out_ref[0, :, pl.ds(_BM // 2, _BM // 2)] = o_t[_OUT:2 * _OUT] + b2t


def _mlp_body_alias(feats_ref, w1d_ref, b1d_ref, w2dt_ref, b2t_ref,
                    prev_ref, out_ref):
    del prev_ref
    _mlp_body(feats_ref, w1d_ref, b1d_ref, w2dt_ref, b2t_ref, out_ref)


def _mlp_chunk(feats2, w1d, b1d, w2dt, b2t, prev=None, b_off=0):
    nb = feats2.shape[0]
    ins = [feats2, w1d, b1d, w2dt, b2t]
    in_specs = [
        pl.BlockSpec((1, _BM // 2, 2 * _FDIM), lambda b, j: (b, j, 0)),
        pl.BlockSpec((2 * _FDIM, 2 * _HIDDEN), lambda b, j: (0, 0)),
        pl.BlockSpec((1, 2 * _HIDDEN), lambda b, j: (0, 0)),
        pl.BlockSpec((2 * _OUT, 2 * _HIDDEN), lambda b, j: (0, 0)),
        pl.BlockSpec((_OUT, 1), lambda b, j: (0, 0)),
    ]
    kwargs = {}
    body = _mlp_body
    if prev is not None:
        ins.append(prev)
        in_specs.append(pl.BlockSpec((1, 8, 128), lambda b, j: (0, 0, 0)))
        kwargs["input_output_aliases"] = {5: 0}
        body = _mlp_body_alias
    return pl.pallas_call(
        body,
        grid=(nb, _N // _BM),
        in_specs=in_specs,
        out_specs=pl.BlockSpec((1, _OUT, _BM),
                               lambda b, j, o=b_off: (b + o, 0, j)),
        out_shape=jax.ShapeDtypeStruct((_B, _OUT, _N), jnp.float32),
        **kwargs,
    )(*ins)


def _mlp_weights(w1, b1, w2, b2):
    z = jnp.zeros((_FDIM, _HIDDEN), jnp.float32)
    w1d = jnp.concatenate(
        [jnp.concatenate([w1, z], axis=1),
         jnp.concatenate([z, w1], axis=1)], axis=0).astype(jnp.bfloat16)
    b1d = jnp.concatenate([b1, b1]).reshape(1, 2 * _HIDDEN)
    zt = jnp.zeros((_OUT, _HIDDEN), jnp.float32)
    w2t = w2.T
    w2dt = jnp.concatenate(
        [jnp.concatenate([w2t, zt], axis=1),
         jnp.concatenate([zt, w2t], axis=1)], axis=0).astype(jnp.bfloat16)
    return w1d, b1d, w2dt, b2.reshape(_OUT, 1)


def kernel(coords, feature_field, W1, b1, W2, b2):
    flat = coords.reshape(-1, 2)
    xs = flat[:, 0]
    ys = flat[:, 1]
    ff_t = jnp.transpose(feature_field, (0, 2, 1))
    table = _prep(ff_t).reshape(_H * _W, _FDIM)
    wts = _mlp_weights(W1, b1, W2, b2)
    h = _M // 2
    bh = _B // 2
    feats_a = _sc_sample_half(xs[:h], ys[:h], table)
    feats_b = _sc_sample_half(xs[h:], ys[h:], table)
    feats_a = feats_a.reshape(bh, _N // 2, 2 * _FDIM)
    feats_b = feats_b.reshape(bh, _N // 2, 2 * _FDIM)
    out_a = _mlp_chunk(feats_a, *wts, prev=None, b_off=0)
    out_t = _mlp_chunk(feats_b, *wts, prev=out_a, b_off=bh)
    return jnp.transpose(out_t, (0, 2, 1))


# speedup vs baseline: 1.2207x; 1.2207x over previous
import functools

import jax
import jax.numpy as jnp
from jax import lax
from jax.experimental import pallas as pl
from jax.experimental.pallas import tpu as pltpu
from jax.experimental.pallas import tpu_sc as plsc

_H, _W, _FDIM = 1024, 1024, 64
_HIDDEN, _OUT = 128, 64
_B, _N = 16, 65536
_M = _B * _N

_NC, _NS, _L = 2, 16, 16
_NW = _NC * _NS
_PW = _M // _NW
_BLK = 128
_NB = _PW // _BLK

_mesh = plsc.VectorSubcoreMesh(core_axis_name="c", subcore_axis_name="s")


def _make_sc_sample(m_pts):
  pw = m_pts // _NW
  nb = pw // _BLK

  @functools.partial(
    pl.kernel,
    mesh=_mesh,
    compiler_params=pltpu.CompilerParams(use_tc_tiling_on_sc=False),
    out_type=jax.ShapeDtypeStruct((m_pts // 2, 2 * _FDIM), jnp.float32),
    scratch_types=[
        pltpu.VMEM((2, _BLK), jnp.float32),
        pltpu.VMEM((2, _BLK), jnp.float32),
        pltpu.VMEM((2, 4, _BLK), jnp.int32),
        pltpu.VMEM((2, 4, _BLK), jnp.float32),
        pltpu.VMEM((2, 4 * _BLK, _FDIM), jnp.float32),
        pltpu.VMEM((2, _BLK, _FDIM), jnp.float32),
        pltpu.SemaphoreType.DMA,
        pltpu.SemaphoreType.DMA,
        pltpu.SemaphoreType.DMA,
        pltpu.SemaphoreType.DMA,
        pltpu.SemaphoreType.DMA,
        pltpu.SemaphoreType.DMA,
    ],
)
  def _sc_sample(xs_hbm, ys_hbm, table_hbm, out_hbm,
                 xs_v, ys_v, idx_v, w_v, rows_v, feats_v,
                 semc0, semc1, semg0, semg1, sems0, sems1):
      wid = lax.axis_index("s") * _NC + lax.axis_index("c")
      base = wid * pw
      semc = (semc0, semc1)
      semg = (semg0, semg1)
      sems = (sems0, sems1)

      def fire_coords(bi, par):
          off = base + bi * _BLK
          pltpu.async_copy(xs_hbm.at[pl.ds(off, _BLK)], xs_v.at[par], semc[par])
          pltpu.async_copy(ys_hbm.at[pl.ds(off, _BLK)], ys_v.at[par], semc[par])

      def wait_coords(par):
          pltpu.make_async_copy(
              xs_hbm.at[pl.ds(0, _BLK)], xs_v.at[par], semc[par]).wait()
          pltpu.make_async_copy(
              ys_hbm.at[pl.ds(0, _BLK)], ys_v.at[par], semc[par]).wait()

      def compute_idx(par):
          for j in range(_BLK // _L):
              sl = pl.ds(j * _L, _L)
              x = xs_v[par, sl] * jnp.float32(_W - 1)
              y = ys_v[par, sl] * jnp.float32(_H - 1)
              xi = jnp.minimum(x.astype(jnp.int32), _W - 2)
              yi = jnp.minimum(y.astype(jnp.int32), _H - 2)
              fx = x - xi.astype(jnp.float32)
              fy = y - yi.astype(jnp.float32)
              gx = 1.0 - fx
              gy = 1.0 - fy
              b00 = yi * _W + xi
              s0 = jnp.where(yi >= _H // 2, _H * _W - 1, 0)
              s1 = jnp.where(yi >= _H // 2 - 1, _H * _W - 1, 0)
              r00 = 2 * b00 - s0
              r10 = 2 * (b00 + _W) - s1
              idx_v[par, 0, sl] = r00
              idx_v[par, 1, sl] = r00 + 2
              idx_v[par, 2, sl] = r10
              idx_v[par, 3, sl] = r10 + 2
              w_v[par, 0, sl] = gx * gy
              w_v[par, 1, sl] = fx * gy
              w_v[par, 2, sl] = gx * fy
              w_v[par, 3, sl] = fx * fy

      def fire_gather(par):
          for c in range(4):
              pltpu.async_copy(table_hbm.at[idx_v.at[par, c]],
                               rows_v.at[par, pl.ds(c * _BLK, _BLK)], semg[par])

      def wait_gather(par):
          for c in range(4):
              pltpu.make_async_copy(
                  table_hbm.at[idx_v.at[par, c]],
                  rows_v.at[par, pl.ds(c * _BLK, _BLK)], semg[par]).wait()

      def blend(par):
          def group_body(g, c2):
              p0 = g * _L
              w00v = w_v[par, 0, pl.ds(p0, _L)]
              w01v = w_v[par, 1, pl.ds(p0, _L)]
              w10v = w_v[par, 2, pl.ds(p0, _L)]
              w11v = w_v[par, 3, pl.ds(p0, _L)]
              for i in range(_L):
                  p = p0 + i
                  w00 = w00v[i]
                  w01 = w01v[i]
                  w10 = w10v[i]
                  w11 = w11v[i]
                  for k in range(_FDIM // _L):
                      ksl = pl.ds(k * _L, _L)
                      acc = rows_v[par, p, ksl] * w00
                      acc = acc + rows_v[par, _BLK + p, ksl] * w01
                      acc = acc + rows_v[par, 2 * _BLK + p, ksl] * w10
                      acc = acc + rows_v[par, 3 * _BLK + p, ksl] * w11
                      feats_v[par, p, ksl] = acc
              return c2

          lax.fori_loop(0, _BLK // _L, group_body, 0)

      def fire_store(bi, par):
          off = base + bi * _BLK
          row_base = (off // 2048) * 1024 + lax.rem(off, 1024)
          half = lax.rem(off // 1024, 2)
          pltpu.async_copy(
              feats_v.at[par],
              out_hbm.at[pl.ds(row_base, _BLK), pl.ds(half * _FDIM, _FDIM)],
              sems[par])

      def wait_store(par):
          pltpu.make_async_copy(
              feats_v.at[par],
              out_hbm.at[pl.ds(0, _BLK), pl.ds(0, _FDIM)], sems[par]).wait()

      fire_coords(0, 0)
      fire_coords(1, 1)
      wait_coords(0)
      compute_idx(0)
      fire_gather(0)
      fire_coords(2, 0)
      wait_coords(1)
      compute_idx(1)
      fire_gather(1)
      fire_coords(3, 1)

      def outer(g, carry):
          for par in (0, 1):
              bi = 2 * g + par
              wait_gather(par)

              @pl.when(bi >= 2)
              def _():
                  wait_store(par)

              blend(par)
              fire_store(bi, par)

              @pl.when(bi + 2 < nb)
              def _():
                  wait_coords(par)
                  compute_idx(par)
                  fire_gather(par)

              @pl.when(bi + 4 < nb)
              def _():
                  fire_coords(bi + 4, par)

          return carry

      lax.fori_loop(0, nb // 2, outer, 0)
      wait_store(0)
      wait_store(1)

  return _sc_sample


_sc_sample_half = _make_sc_sample(_M // 2)


_PY = 8


def _prep_body(ff_lo_ref, ff_hi_ref, out_ref):
    for r in range(_PY):
        out_ref[r, :, pl.ds(0, _FDIM)] = ff_lo_ref[r].T
        out_ref[r, :, pl.ds(_FDIM, _FDIM)] = ff_hi_ref[r].T


def _prep(ff_t):
    return pl.pallas_call(
        _prep_body,
        grid=(_H // 2 // _PY,),
        in_specs=[
            pl.BlockSpec((_PY, _FDIM, _W), lambda y: (y, 0, 0)),
            pl.BlockSpec((_PY, _FDIM, _W),
                         lambda y: (y + _H // 2 // _PY, 0, 0)),
        ],
        out_specs=pl.BlockSpec((_PY, _W, 2 * _FDIM), lambda y: (y, 0, 0)),
        out_shape=jax.ShapeDtypeStruct((_H // 2, _W, 2 * _FDIM), jnp.float32),
    )(ff_t, ff_t)


_BM = 2048


def _mlp_body(feats_ref, w1d_ref, b1d_ref, w2dt_ref, b2t_ref, out_ref):
    f2 = feats_ref[0].astype(jnp.bfloat16)
    h = jnp.dot(f2, w1d_ref[...],
                preferred_element_type=jnp.float32) + b1d_ref[...]
    h = jnp.maximum(h, 0.0).astype(jnp.bfloat16)
    o_t = lax.dot_general(w2dt_ref[...], h, (((1,), (1,)), ((), ())),
                          preferred_element_type=jnp.float32)
    b2t = b2t_ref[...]
    out_ref[0, :, pl.ds(0, _BM // 2)] = o_t[0:_OUT] + b2t
    out_ref[0, :, pl.ds(_BM // 2, _BM // 2)] = o_t[_OUT:2 * _OUT] + b2t


def _mlp_body_alias(feats_ref, w1d_ref, b1d_ref, w2dt_ref, b2t_ref,
                    prev_ref, out_ref):
    del prev_ref
    _mlp_body(feats_ref, w1d_ref, b1d_ref, w2dt_ref, b2t_ref, out_ref)


def _mlp_chunk(feats2, w1d, b1d, w2dt, b2t, prev=None, b_off=0):
    nb = feats2.shape[0]
    ins = [feats2, w1d, b1d, w2dt, b2t]
    in_specs = [
        pl.BlockSpec((1, _BM // 2, 2 * _FDIM), lambda b, j: (b, j, 0)),
        pl.BlockSpec((2 * _FDIM, 2 * _HIDDEN), lambda b, j: (0, 0)),
        pl.BlockSpec((1, 2 * _HIDDEN), lambda b, j: (0, 0)),
        pl.BlockSpec((2 * _OUT, 2 * _HIDDEN), lambda b, j: (0, 0)),
        pl.BlockSpec((_OUT, 1), lambda b, j: (0, 0)),
    ]
    kwargs = {}
    body = _mlp_body
    if prev is not None:
        ins.append(prev)
        in_specs.append(pl.BlockSpec((1, 8, 128), lambda b, j: (0, 0, 0)))
        kwargs["input_output_aliases"] = {5: 0}
        body = _mlp_body_alias
    return pl.pallas_call(
        body,
        grid=(nb, _N // _BM),
        in_specs=in_specs,
        out_specs=pl.BlockSpec((1, _OUT, _BM),
                               lambda b, j, o=b_off: (b + o, 0, j)),
        out_shape=jax.ShapeDtypeStruct((_B, _OUT, _N), jnp.float32),
        **kwargs,
    )(*ins)


def _mlp_weights(w1, b1, w2, b2):
    z = jnp.zeros((_FDIM, _HIDDEN), jnp.float32)
    w1d = jnp.concatenate(
        [jnp.concatenate([w1, z], axis=1),
         jnp.concatenate([z, w1], axis=1)], axis=0).astype(jnp.bfloat16)
    b1d = jnp.concatenate([b1, b1]).reshape(1, 2 * _HIDDEN)
    zt = jnp.zeros((_OUT, _HIDDEN), jnp.float32)
    w2t = w2.T
    w2dt = jnp.concatenate(
        [jnp.concatenate([w2t, zt], axis=1),
         jnp.concatenate([zt, w2t], axis=1)], axis=0).astype(jnp.bfloat16)
    return w1d, b1d, w2dt, b2.reshape(_OUT, 1)


def kernel(coords, feature_field, W1, b1, W2, b2):
    flat = coords.reshape(-1, 2)
    xs = flat[:, 0]
    ys = flat[:, 1]
    ff_t = jnp.transpose(feature_field, (0, 2, 1))
    table = _prep(ff_t).reshape(_H * _W, _FDIM)
    wts = _mlp_weights(W1, b1, W2, b2)
    h = _M // 2
    bh = _B // 2
    feats_a = _sc_sample_half(xs[:h], ys[:h], table)
    feats_b = _sc_sample_half(xs[h:], ys[h:], table)
    feats_a = feats_a.reshape(bh, _N // 2, 2 * _FDIM)
    feats_b = feats_b.reshape(bh, _N // 2, 2 * _FDIM)
    out_a = _mlp_chunk(feats_a, *wts, prev=None, b_off=0)
    out_t = _mlp_chunk(feats_b, *wts, prev=out_a, b_off=bh)
    return jnp.transpose(out_t, (0, 2, 1))


# speedup vs baseline: 1.3651x; 1.1183x over previous
import functools

import jax
import jax.numpy as jnp
from jax import lax
from jax.experimental import pallas as pl
from jax.experimental.pallas import tpu as pltpu
from jax.experimental.pallas import tpu_sc as plsc

_H, _W, _FDIM = 1024, 1024, 64
_HIDDEN, _OUT = 128, 64
_B, _N = 16, 65536
_M = _B * _N

_NC, _NS, _L = 2, 16, 16
_NW = _NC * _NS
_PW = _M // _NW
_BLK = 128
_NB = _PW // _BLK

_mesh = plsc.VectorSubcoreMesh(core_axis_name="c", subcore_axis_name="s")


def _make_sc_sample(m_pts):
  pw = m_pts // _NW
  nb = pw // _BLK

  @functools.partial(
    pl.kernel,
    mesh=_mesh,
    compiler_params=pltpu.CompilerParams(use_tc_tiling_on_sc=False),
    out_type=jax.ShapeDtypeStruct((m_pts // 2, 2 * _FDIM), jnp.float32),
    scratch_types=[
        pltpu.VMEM((2, _BLK), jnp.float32),
        pltpu.VMEM((2, _BLK), jnp.float32),
        pltpu.VMEM((2, 4, _BLK), jnp.int32),
        pltpu.VMEM((2, 4, _BLK), jnp.float32),
        pltpu.VMEM((2, 4 * _BLK, _FDIM), jnp.float32),
        pltpu.VMEM((2, _BLK, _FDIM), jnp.float32),
        pltpu.SemaphoreType.DMA,
        pltpu.SemaphoreType.DMA,
        pltpu.SemaphoreType.DMA,
        pltpu.SemaphoreType.DMA,
        pltpu.SemaphoreType.DMA,
        pltpu.SemaphoreType.DMA,
    ],
)
  def _sc_sample(xs_hbm, ys_hbm, table_hbm, out_hbm,
                 xs_v, ys_v, idx_v, w_v, rows_v, feats_v,
                 semc0, semc1, semg0, semg1, sems0, sems1):
      wid = lax.axis_index("s") * _NC + lax.axis_index("c")
      base = wid * pw
      semc = (semc0, semc1)
      semg = (semg0, semg1)
      sems = (sems0, sems1)

      def fire_coords(bi, par):
          off = base + bi * _BLK
          pltpu.async_copy(xs_hbm.at[pl.ds(off, _BLK)], xs_v.at[par], semc[par])
          pltpu.async_copy(ys_hbm.at[pl.ds(off, _BLK)], ys_v.at[par], semc[par])

      def wait_coords(par):
          pltpu.make_async_copy(
              xs_hbm.at[pl.ds(0, _BLK)], xs_v.at[par], semc[par]).wait()
          pltpu.make_async_copy(
              ys_hbm.at[pl.ds(0, _BLK)], ys_v.at[par], semc[par]).wait()

      def compute_idx(par):
          for j in range(_BLK // _L):
              sl = pl.ds(j * _L, _L)
              x = xs_v[par, sl] * jnp.float32(_W - 1)
              y = ys_v[par, sl] * jnp.float32(_H - 1)
              xi = jnp.minimum(x.astype(jnp.int32), _W - 2)
              yi = jnp.minimum(y.astype(jnp.int32), _H - 2)
              fx = x - xi.astype(jnp.float32)
              fy = y - yi.astype(jnp.float32)
              gx = 1.0 - fx
              gy = 1.0 - fy
              b00 = yi * _W + xi
              s0 = jnp.where(yi >= _H // 2, _H * _W - 1, 0)
              s1 = jnp.where(yi >= _H // 2 - 1, _H * _W - 1, 0)
              r00 = 2 * b00 - s0
              r10 = 2 * (b00 + _W) - s1
              idx_v[par, 0, sl] = r00
              idx_v[par, 1, sl] = r00 + 2
              idx_v[par, 2, sl] = r10
              idx_v[par, 3, sl] = r10 + 2
              w_v[par, 0, sl] = gx * gy
              w_v[par, 1, sl] = fx * gy
              w_v[par, 2, sl] = gx * fy
              w_v[par, 3, sl] = fx * fy

      def fire_gather(par):
          for c in range(4):
              pltpu.async_copy(table_hbm.at[idx_v.at[par, c]],
                               rows_v.at[par, pl.ds(c * _BLK, _BLK)], semg[par])

      def wait_gather(par):
          for c in range(4):
              pltpu.make_async_copy(
                  table_hbm.at[idx_v.at[par, c]],
                  rows_v.at[par, pl.ds(c * _BLK, _BLK)], semg[par]).wait()

      def blend(par):
          def group_body(g, c2):
              p0 = g * _L
              w00v = w_v[par, 0, pl.ds(p0, _L)]
              w01v = w_v[par, 1, pl.ds(p0, _L)]
              w10v = w_v[par, 2, pl.ds(p0, _L)]
              w11v = w_v[par, 3, pl.ds(p0, _L)]
              for i in range(_L):
                  p = p0 + i
                  w00 = w00v[i]
                  w01 = w01v[i]
                  w10 = w10v[i]
                  w11 = w11v[i]
                  for k in range(_FDIM // _L):
                      ksl = pl.ds(k * _L, _L)
                      acc = rows_v[par, p, ksl] * w00
                      acc = acc + rows_v[par, _BLK + p, ksl] * w01
                      acc = acc + rows_v[par, 2 * _BLK + p, ksl] * w10
                      acc = acc + rows_v[par, 3 * _BLK + p, ksl] * w11
                      feats_v[par, p, ksl] = acc
              return c2

          lax.fori_loop(0, _BLK // _L, group_body, 0)

      def fire_store(bi, par):
          off = base + bi * _BLK
          row_base = (off // 2048) * 1024 + lax.rem(off, 1024)
          half = lax.rem(off // 1024, 2)
          pltpu.async_copy(
              feats_v.at[par],
              out_hbm.at[pl.ds(row_base, _BLK), pl.ds(half * _FDIM, _FDIM)],
              sems[par])

      def wait_store(par):
          pltpu.make_async_copy(
              feats_v.at[par],
              out_hbm.at[pl.ds(0, _BLK), pl.ds(0, _FDIM)], sems[par]).wait()

      fire_coords(0, 0)
      fire_coords(1, 1)
      wait_coords(0)
      compute_idx(0)
      fire_gather(0)
      fire_coords(2, 0)
      wait_coords(1)
      compute_idx(1)
      fire_gather(1)
      fire_coords(3, 1)

      def outer(g, carry):
          for par in (0, 1):
              bi = 2 * g + par
              wait_gather(par)

              @pl.when(bi >= 2)
              def _():
                  wait_store(par)

              blend(par)
              fire_store(bi, par)

              @pl.when(bi + 2 < nb)
              def _():
                  wait_coords(par)
                  compute_idx(par)
                  fire_gather(par)

              @pl.when(bi + 4 < nb)
              def _():
                  fire_coords(bi + 4, par)

          return carry

      lax.fori_loop(0, nb // 2, outer, 0)
      wait_store(0)
      wait_store(1)

  return _sc_sample


_sc_sample_q = _make_sc_sample(_M // 4)


_PY = 8


def _prep_body(ff_lo_ref, ff_hi_ref, out_ref):
    for r in range(_PY):
        out_ref[r, :, pl.ds(0, _FDIM)] = ff_lo_ref[r].T
        out_ref[r, :, pl.ds(_FDIM, _FDIM)] = ff_hi_ref[r].T


def _prep(ff_t):
    return pl.pallas_call(
        _prep_body,
        grid=(_H // 2 // _PY,),
        in_specs=[
            pl.BlockSpec((_PY, _FDIM, _W), lambda y: (y, 0, 0)),
            pl.BlockSpec((_PY, _FDIM, _W),
                         lambda y: (y + _H // 2 // _PY, 0, 0)),
        ],
        out_specs=pl.BlockSpec((_PY, _W, 2 * _FDIM), lambda y: (y, 0, 0)),
        out_shape=jax.ShapeDtypeStruct((_H // 2, _W, 2 * _FDIM), jnp.float32),
    )(ff_t, ff_t)


_BM = 8192


def _mlp_body(feats_ref, w1d_ref, b1d_ref, w2dt_ref, b2t_ref, out_ref):
    f2 = feats_ref[0].astype(jnp.bfloat16)
    h = jnp.dot(f2, w1d_ref[...],
                preferred_element_type=jnp.float32) + b1d_ref[...]
    h = jnp.maximum(h, 0.0).astype(jnp.bfloat16)
    o_t = lax.dot_general(w2dt_ref[...], h, (((1,), (1,)), ((), ())),
                          preferred_element_type=jnp.float32)
    b2t = b2t_ref[...]
    for s2 in range(_BM // 2048):
        seg = o_t[:, s2 * 1024:(s2 + 1) * 1024]
        out_ref[0, :, pl.ds(s2 * 2048, 1024)] = seg[0:_OUT] + b2t
        out_ref[0, :, pl.ds(s2 * 2048 + 1024, 1024)] = seg[_OUT:2 * _OUT] + b2t


def _mlp_body_alias(feats_ref, w1d_ref, b1d_ref, w2dt_ref, b2t_ref,
                    prev_ref, out_ref):
    del prev_ref
    _mlp_body(feats_ref, w1d_ref, b1d_ref, w2dt_ref, b2t_ref, out_ref)


def _mlp_chunk(feats2, w1d, b1d, w2dt, b2t, prev=None, b_off=0):
    nb = feats2.shape[0]
    ins = [feats2, w1d, b1d, w2dt, b2t]
    in_specs = [
        pl.BlockSpec((1, _BM // 2, 2 * _FDIM), lambda b, j: (b, j, 0)),
        pl.BlockSpec((2 * _FDIM, 2 * _HIDDEN), lambda b, j: (0, 0)),
        pl.BlockSpec((1, 2 * _HIDDEN), lambda b, j: (0, 0)),
        pl.BlockSpec((2 * _OUT, 2 * _HIDDEN), lambda b, j: (0, 0)),
        pl.BlockSpec((_OUT, 1), lambda b, j: (0, 0)),
    ]
    kwargs = {}
    body = _mlp_body
    if prev is not None:
        ins.append(prev)
        in_specs.append(pl.BlockSpec((1, 8, 128), lambda b, j: (0, 0, 0)))
        kwargs["input_output_aliases"] = {5: 0}
        body = _mlp_body_alias
    return pl.pallas_call(
        body,
        grid=(nb, _N // _BM),
        in_specs=in_specs,
        out_specs=pl.BlockSpec((1, _OUT, _BM),
                               lambda b, j, o=b_off: (b + o, 0, j)),
        out_shape=jax.ShapeDtypeStruct((_B, _OUT, _N), jnp.float32),
        **kwargs,
    )(*ins)


def _mlp_weights(w1, b1, w2, b2):
    z = jnp.zeros((_FDIM, _HIDDEN), jnp.float32)
    w1d = jnp.concatenate(
        [jnp.concatenate([w1, z], axis=1),
         jnp.concatenate([z, w1], axis=1)], axis=0).astype(jnp.bfloat16)
    b1d = jnp.concatenate([b1, b1]).reshape(1, 2 * _HIDDEN)
    zt = jnp.zeros((_OUT, _HIDDEN), jnp.float32)
    w2t = w2.T
    w2dt = jnp.concatenate(
        [jnp.concatenate([w2t, zt], axis=1),
         jnp.concatenate([zt, w2t], axis=1)], axis=0).astype(jnp.bfloat16)
    return w1d, b1d, w2dt, b2.reshape(_OUT, 1)


def kernel(coords, feature_field, W1, b1, W2, b2):
    flat = coords.reshape(-1, 2)
    xs = flat[:, 0]
    ys = flat[:, 1]
    ff_t = jnp.transpose(feature_field, (0, 2, 1))
    table = _prep(ff_t).reshape(_H * _W, _FDIM)
    wts = _mlp_weights(W1, b1, W2, b2)
    q = _M // 4
    bq = _B // 4
    feats = [
        _sc_sample_q(xs[i * q:(i + 1) * q], ys[i * q:(i + 1) * q],
                     table).reshape(bq, _N // 2, 2 * _FDIM)
        for i in range(4)
    ]
    out_t = None
    for i in range(4):
        out_t = _mlp_chunk(feats[i], *wts, prev=out_t, b_off=i * bq)
    return jnp.transpose(out_t, (0, 2, 1))


# speedup vs baseline: 1.3682x; 1.0023x over previous
import functools

import jax
import jax.numpy as jnp
from jax import lax
from jax.experimental import pallas as pl
from jax.experimental.pallas import tpu as pltpu
from jax.experimental.pallas import tpu_sc as plsc

_H, _W, _FDIM = 1024, 1024, 64
_HIDDEN, _OUT = 128, 64
_B, _N = 16, 65536
_M = _B * _N

_NC, _NS, _L = 2, 16, 16
_NW = _NC * _NS
_PW = _M // _NW
_BLK = 128
_NB = _PW // _BLK

_mesh = plsc.VectorSubcoreMesh(core_axis_name="c", subcore_axis_name="s")


def _make_sc_sample(m_pts):
  pw = m_pts // _NW
  nb = pw // _BLK

  @functools.partial(
    pl.kernel,
    mesh=_mesh,
    compiler_params=pltpu.CompilerParams(use_tc_tiling_on_sc=False),
    out_type=jax.ShapeDtypeStruct((m_pts // 2, 2 * _FDIM), jnp.float32),
    scratch_types=[
        pltpu.VMEM((2, _BLK), jnp.float32),
        pltpu.VMEM((2, _BLK), jnp.float32),
        pltpu.VMEM((2, 4, _BLK), jnp.int32),
        pltpu.VMEM((2, 4, _BLK), jnp.float32),
        pltpu.VMEM((2, 4 * _BLK, _FDIM), jnp.float32),
        pltpu.VMEM((2, _BLK, _FDIM), jnp.float32),
        pltpu.SemaphoreType.DMA,
        pltpu.SemaphoreType.DMA,
        pltpu.SemaphoreType.DMA,
        pltpu.SemaphoreType.DMA,
        pltpu.SemaphoreType.DMA,
        pltpu.SemaphoreType.DMA,
    ],
)
  def _sc_sample(xs_hbm, ys_hbm, table_hbm, out_hbm,
                 xs_v, ys_v, idx_v, w_v, rows_v, feats_v,
                 semc0, semc1, semg0, semg1, sems0, sems1):
      wid = lax.axis_index("s") * _NC + lax.axis_index("c")
      base = wid * pw
      semc = (semc0, semc1)
      semg = (semg0, semg1)
      sems = (sems0, sems1)

      def fire_coords(bi, par):
          off = base + bi * _BLK
          pltpu.async_copy(xs_hbm.at[pl.ds(off, _BLK)], xs_v.at[par], semc[par])
          pltpu.async_copy(ys_hbm.at[pl.ds(off, _BLK)], ys_v.at[par], semc[par])

      def wait_coords(par):
          pltpu.make_async_copy(
              xs_hbm.at[pl.ds(0, _BLK)], xs_v.at[par], semc[par]).wait()
          pltpu.make_async_copy(
              ys_hbm.at[pl.ds(0, _BLK)], ys_v.at[par], semc[par]).wait()

      def compute_idx(par):
          for j in range(_BLK // _L):
              sl = pl.ds(j * _L, _L)
              x = xs_v[par, sl] * jnp.float32(_W - 1)
              y = ys_v[par, sl] * jnp.float32(_H - 1)
              xi = jnp.minimum(x.astype(jnp.int32), _W - 2)
              yi = jnp.minimum(y.astype(jnp.int32), _H - 2)
              fx = x - xi.astype(jnp.float32)
              fy = y - yi.astype(jnp.float32)
              gx = 1.0 - fx
              gy = 1.0 - fy
              b00 = yi * _W + xi
              s0 = jnp.where(yi >= _H // 2, _H * _W - 1, 0)
              s1 = jnp.where(yi >= _H // 2 - 1, _H * _W - 1, 0)
              r00 = 2 * b00 - s0
              r10 = 2 * (b00 + _W) - s1
              idx_v[par, 0, sl] = r00
              idx_v[par, 1, sl] = r00 + 2
              idx_v[par, 2, sl] = r10
              idx_v[par, 3, sl] = r10 + 2
              w_v[par, 0, sl] = gx * gy
              w_v[par, 1, sl] = fx * gy
              w_v[par, 2, sl] = gx * fy
              w_v[par, 3, sl] = fx * fy

      def fire_gather(par):
          for c in range(4):
              pltpu.async_copy(table_hbm.at[idx_v.at[par, c]],
                               rows_v.at[par, pl.ds(c * _BLK, _BLK)], semg[par])

      def wait_gather(par):
          for c in range(4):
              pltpu.make_async_copy(
                  table_hbm.at[idx_v.at[par, c]],
                  rows_v.at[par, pl.ds(c * _BLK, _BLK)], semg[par]).wait()

      def blend(par):
          def group_body(g, c2):
              p0 = g * _L
              w00v = w_v[par, 0, pl.ds(p0, _L)]
              w01v = w_v[par, 1, pl.ds(p0, _L)]
              w10v = w_v[par, 2, pl.ds(p0, _L)]
              w11v = w_v[par, 3, pl.ds(p0, _L)]
              for i in range(_L):
                  p = p0 + i
                  w00 = w00v[i]
                  w01 = w01v[i]
                  w10 = w10v[i]
                  w11 = w11v[i]
                  for k in range(_FDIM // _L):
                      ksl = pl.ds(k * _L, _L)
                      acc = rows_v[par, p, ksl] * w00
                      acc = acc + rows_v[par, _BLK + p, ksl] * w01
                      acc = acc + rows_v[par, 2 * _BLK + p, ksl] * w10
                      acc = acc + rows_v[par, 3 * _BLK + p, ksl] * w11
                      feats_v[par, p, ksl] = acc
              return c2

          lax.fori_loop(0, _BLK // _L, group_body, 0)

      def fire_store(bi, par):
          off = base + bi * _BLK
          row_base = (off // 2048) * 1024 + lax.rem(off, 1024)
          half = lax.rem(off // 1024, 2)
          pltpu.async_copy(
              feats_v.at[par],
              out_hbm.at[pl.ds(row_base, _BLK), pl.ds(half * _FDIM, _FDIM)],
              sems[par])

      def wait_store(par):
          pltpu.make_async_copy(
              feats_v.at[par],
              out_hbm.at[pl.ds(0, _BLK), pl.ds(0, _FDIM)], sems[par]).wait()

      fire_coords(0, 0)
      fire_coords(1, 1)
      wait_coords(0)
      compute_idx(0)
      fire_gather(0)
      fire_coords(2, 0)
      wait_coords(1)
      compute_idx(1)
      fire_gather(1)
      fire_coords(3, 1)

      def outer(g, carry):
          for par in (0, 1):
              bi = 2 * g + par
              wait_gather(par)

              @pl.when(bi >= 2)
              def _():
                  wait_store(par)

              blend(par)
              fire_store(bi, par)

              @pl.when(bi + 2 < nb)
              def _():
                  wait_coords(par)
                  compute_idx(par)
                  fire_gather(par)

              @pl.when(bi + 4 < nb)
              def _():
                  fire_coords(bi + 4, par)

          return carry

      lax.fori_loop(0, nb // 2, outer, 0)
      wait_store(0)
      wait_store(1)

  return _sc_sample


_CHUNKS = (6, 4, 4, 2)
_sc_samplers = {s: _make_sc_sample(s * _N) for s in set(_CHUNKS)}


_PY = 8


def _prep_body(ff_lo_ref, ff_hi_ref, out_ref):
    eye = jnp.eye(_FDIM, dtype=jnp.float32)
    dn = (((0,), (0,)), ((), ()))
    for r in range(_PY):
        if r % 2 == 0:
            ta = ff_lo_ref[r].T
            tb = ff_hi_ref[r].T
        else:
            ta = lax.dot_general(ff_lo_ref[r], eye, dn,
                                 preferred_element_type=jnp.float32)
            tb = lax.dot_general(ff_hi_ref[r], eye, dn,
                                 preferred_element_type=jnp.float32)
        out_ref[r, :, pl.ds(0, _FDIM)] = ta
        out_ref[r, :, pl.ds(_FDIM, _FDIM)] = tb


def _prep(ff_t):
    return pl.pallas_call(
        _prep_body,
        grid=(_H // 2 // _PY,),
        in_specs=[
            pl.BlockSpec((_PY, _FDIM, _W), lambda y: (y, 0, 0)),
            pl.BlockSpec((_PY, _FDIM, _W),
                         lambda y: (y + _H // 2 // _PY, 0, 0)),
        ],
        out_specs=pl.BlockSpec((_PY, _W, 2 * _FDIM), lambda y: (y, 0, 0)),
        out_shape=jax.ShapeDtypeStruct((_H // 2, _W, 2 * _FDIM), jnp.float32),
    )(ff_t, ff_t)


_BM = 8192


def _mlp_body(feats_ref, w1d_ref, b1d_ref, w2dt_ref, b2t_ref, out_ref):
    f2 = feats_ref[0].astype(jnp.bfloat16)
    h = jnp.dot(f2, w1d_ref[...],
                preferred_element_type=jnp.float32) + b1d_ref[...]
    h = jnp.maximum(h, 0.0).astype(jnp.bfloat16)
    o_t = lax.dot_general(w2dt_ref[...], h, (((1,), (1,)), ((), ())),
                          preferred_element_type=jnp.float32)
    b2t = b2t_ref[...]
    for s2 in range(_BM // 2048):
        seg = o_t[:, s2 * 1024:(s2 + 1) * 1024]
        out_ref[0, :, pl.ds(s2 * 2048, 1024)] = seg[0:_OUT] + b2t
        out_ref[0, :, pl.ds(s2 * 2048 + 1024, 1024)] = seg[_OUT:2 * _OUT] + b2t


def _mlp_body_alias(feats_ref, w1d_ref, b1d_ref, w2dt_ref, b2t_ref,
                    prev_ref, out_ref):
    del prev_ref
    _mlp_body(feats_ref, w1d_ref, b1d_ref, w2dt_ref, b2t_ref, out_ref)


def _mlp_chunk(feats2, w1d, b1d, w2dt, b2t, prev=None, b_off=0):
    nb = feats2.shape[0]
    ins = [feats2, w1d, b1d, w2dt, b2t]
    in_specs = [
        pl.BlockSpec((1, _BM // 2, 2 * _FDIM), lambda b, j: (b, j, 0)),
        pl.BlockSpec((2 * _FDIM, 2 * _HIDDEN), lambda b, j: (0, 0)),
        pl.BlockSpec((1, 2 * _HIDDEN), lambda b, j: (0, 0)),
        pl.BlockSpec((2 * _OUT, 2 * _HIDDEN), lambda b, j: (0, 0)),
        pl.BlockSpec((_OUT, 1), lambda b, j: (0, 0)),
    ]
    kwargs = {}
    body = _mlp_body
    if prev is not None:
        ins.append(prev)
        in_specs.append(pl.BlockSpec((1, 8, 128), lambda b, j: (0, 0, 0)))
        kwargs["input_output_aliases"] = {5: 0}
        body = _mlp_body_alias
    return pl.pallas_call(
        body,
        grid=(nb, _N // _BM),
        in_specs=in_specs,
        out_specs=pl.BlockSpec((1, _OUT, _BM),
                               lambda b, j, o=b_off: (b + o, 0, j)),
        out_shape=jax.ShapeDtypeStruct((_B, _OUT, _N), jnp.float32),
        **kwargs,
    )(*ins)


def _mlp_weights(w1, b1, w2, b2):
    z = jnp.zeros((_FDIM, _HIDDEN), jnp.float32)
    w1d = jnp.concatenate(
        [jnp.concatenate([w1, z], axis=1),
         jnp.concatenate([z, w1], axis=1)], axis=0).astype(jnp.bfloat16)
    b1d = jnp.concatenate([b1, b1]).reshape(1, 2 * _HIDDEN)
    zt = jnp.zeros((_OUT, _HIDDEN), jnp.float32)
    w2t = w2.T
    w2dt = jnp.concatenate(
        [jnp.concatenate([w2t, zt], axis=1),
         jnp.concatenate([zt, w2t], axis=1)], axis=0).astype(jnp.bfloat16)
    return w1d, b1d, w2dt, b2.reshape(_OUT, 1)


def kernel(coords, feature_field, W1, b1, W2, b2):
    flat = coords.reshape(-1, 2)
    xs = flat[:, 0]
    ys = flat[:, 1]
    ff_t = jnp.transpose(feature_field, (0, 2, 1))
    table = _prep(ff_t).reshape(_H * _W, _FDIM)
    wts = _mlp_weights(W1, b1, W2, b2)
    feats = []
    p0 = 0
    for s in _CHUNKS:
        npts = s * _N
        feats.append(
            _sc_samplers[s](xs[p0:p0 + npts], ys[p0:p0 + npts],
                            table).reshape(s, _N // 2, 2 * _FDIM))
        p0 += npts
    out_t = None
    b0 = 0
    for s, f in zip(_CHUNKS, feats):
        out_t = _mlp_chunk(f, *wts, prev=out_t, b_off=b0)
        b0 += s
    return jnp.transpose(out_t, (0, 2, 1))
